# baseline (speedup 1.0000x reference)
import functools

import jax
import jax.numpy as jnp
from jax import lax
from jax.experimental import pallas as pl
from jax.experimental.pallas import tpu as pltpu
from jax.experimental.pallas import tpu_sc as plsc

B = 4096
WLEN = 200
ELEN = 50
EPAD = 64
WSTRIDE = 256
ESTRIDE = 128
DIM = 64
NUM_CLASSES = 16

NC = 2
NS = 16
NW = NC * NS
BPW = B // NW
NCH = BPW


def _rsqrt16(x):
    xi = plsc.bitcast(x, jnp.int32)
    y = plsc.bitcast(jnp.int32(0x5F3759DF) - (xi >> 1), jnp.float32)
    for _ in range(3):
        y = y * (1.5 - 0.5 * x * y * y)
    return y


def _sc_forward(wt2, et2, widx_flat, eidx_flat, pp_flat, asc):
    mesh = plsc.VectorSubcoreMesh(core_axis_name="c", subcore_axis_name="s",
                                  num_cores=NC, num_subcores=NS)

    @functools.partial(
        pl.kernel,
        out_type=(
            jax.ShapeDtypeStruct((B, 128), jnp.float32),
            jax.ShapeDtypeStruct((B, 128), jnp.float32),
        ),
        mesh=mesh,
        scratch_types=[
            pltpu.VMEM((WSTRIDE,), jnp.int32),
            pltpu.VMEM((WSTRIDE,), jnp.int32),
            pltpu.VMEM((WLEN,), jnp.int32),
            pltpu.VMEM((WLEN,), jnp.int32),
            pltpu.VMEM((WLEN + 16,), jnp.int32),
            pltpu.VMEM((WLEN + 16,), jnp.int32),
            pltpu.VMEM((ESTRIDE,), jnp.int32),
            pltpu.VMEM((ESTRIDE,), jnp.int32),
            pltpu.VMEM((EPAD,), jnp.int32),
            pltpu.VMEM((EPAD,), jnp.int32),
            pltpu.VMEM((EPAD,), jnp.int32),
            pltpu.VMEM((EPAD,), jnp.int32),
            pltpu.VMEM((EPAD + 16,), jnp.int32),
            pltpu.VMEM((EPAD + 16,), jnp.int32),
            pltpu.VMEM((WLEN, 128), jnp.float32),
            pltpu.VMEM((WLEN, 128), jnp.float32),
            pltpu.VMEM((EPAD, 128), jnp.float32),
            pltpu.VMEM((EPAD, 128), jnp.float32),
            pltpu.VMEM((BPW * ESTRIDE,), jnp.float32),
            pltpu.VMEM((16,), jnp.float32),
            pltpu.VMEM((DIM + 16,), jnp.float32),
            pltpu.VMEM((EPAD + 16,), jnp.float32),
            pltpu.VMEM((BPW, 128), jnp.float32),
            pltpu.VMEM((BPW, 128), jnp.float32),
            pltpu.SemaphoreType.DMA,
            pltpu.SemaphoreType.DMA,
            pltpu.SemaphoreType.DMA,
            pltpu.SemaphoreType.DMA,
        ],
        compiler_params=pltpu.CompilerParams(use_tc_tiling_on_sc=True,
                                             needs_layout_passes=False),
    )
    def k(wt2_h, et2_h, widx_h, eidx_h, pp_h, asc_h, ws_out, feat_out,
          widx0, widx1, wpair0, wpair1, whoff0, whoff1,
          eldg0, eldg1, eid0, eid1, epair0, epair1, ehoff0, ehoff1,
          wrows0, wrows1, erows0, erows1,
          pp_v, asc_v, wnv_v, att_v, ws_acc, feat_acc,
          semi0, semi1, semg0, semg1):
        w = lax.axis_index("s") * NC + lax.axis_index("c")
        woff = w * BPW
        widx = [widx0, widx1]
        wpair = [wpair0, wpair1]
        whoff = [whoff0, whoff1]
        eldg = [eldg0, eldg1]
        eid = [eid0, eid1]
        epair = [epair0, epair1]
        ehoff = [ehoff0, ehoff1]
        wrows = [wrows0, wrows1]
        erows = [erows0, erows1]
        semi = [semi0, semi1]
        semg = [semg0, semg1]
        zero16 = jnp.zeros((16,), jnp.float32)
        iota16 = lax.iota(jnp.int32, 16)

        pltpu.sync_copy(pp_h.at[pl.ds(woff * ESTRIDE, BPW * ESTRIDE)], pp_v)
        pltpu.sync_copy(asc_h, asc_v)
        ascv = asc_v[pl.ds(0, 16)]
        s_w0 = ascv[0]
        s_w1 = ascv[1]
        s_b = ascv[2]

        def zrow(r, carry):
            for j in range(4):
                ws_acc[r, pl.ds(DIM + 16 * j, 16)] = zero16
                feat_acc[r, pl.ds(DIM + 16 * j, 16)] = zero16
            return carry
        lax.fori_loop(0, BPW, zrow, 0)

        def fetch_idx(c, p):
            pltpu.async_copy(widx_h.at[pl.ds((woff + c) * WSTRIDE, WSTRIDE)],
                             widx[p], semi[p])
            pltpu.async_copy(eidx_h.at[pl.ds((woff + c) * ESTRIDE, ESTRIDE)],
                             eldg[p], semi[p])

        def wait_idx(p):
            pltpu.make_async_copy(widx_h.at[pl.ds(0, WSTRIDE)],
                                  widx[p], semi[p]).wait()
            pltpu.make_async_copy(eidx_h.at[pl.ds(0, ESTRIDE)],
                                  eldg[p], semi[p]).wait()

        def prep(p):
            for kk in range(13):
                off = 184 if kk == 12 else 16 * kk
                v = widx[p][pl.ds(off, 16)]
                wpair[p][pl.ds(off, 16)] = v >> 1
                whoff[p][pl.ds(off, 16)] = (v & 1) << 6
            for kk in range(4):
                v = eldg[p][pl.ds(16 * kk, 16)]
                eid[p][pl.ds(16 * kk, 16)] = v
                epair[p][pl.ds(16 * kk, 16)] = v >> 1
                ehoff[p][pl.ds(16 * kk, 16)] = (v & 1) << 6

        def issue_gathers(p):
            pltpu.async_copy(wt2_h.at[wpair[p].at[pl.ds(0, 128)]],
                             wrows[p].at[pl.ds(0, 128)], semg[p])
            pltpu.async_copy(wt2_h.at[wpair[p].at[pl.ds(128, 72)]],
                             wrows[p].at[pl.ds(128, 72)], semg[p])
            pltpu.async_copy(et2_h.at[epair[p]], erows[p], semg[p])

        def wait_gathers(p):
            pltpu.make_async_copy(wt2_h.at[pl.ds(0, WLEN)],
                                  wrows[p], semg[p]).wait()
            pltpu.make_async_copy(et2_h.at[pl.ds(0, EPAD)],
                                  erows[p], semg[p]).wait()

        def compute(c, p):
            def rbody(r, accs):
                h = pl.multiple_of(whoff[p][pl.ds(r, 16)][0], 64)
                return tuple(accs[j] + wrows[p][r, pl.ds(h + 16 * j, 16)]
                             for j in range(4))
            ws = lax.fori_loop(0, WLEN, rbody, (zero16,) * 4, unroll=2)
            n2 = ws[0] * ws[0] + ws[1] * ws[1] + ws[2] * ws[2] + ws[3] * ws[3]
            rsv = _rsqrt16(jnp.maximum(jnp.full((16,), jnp.sum(n2)), 1e-24))
            for j in range(4):
                ws_acc[c, pl.ds(16 * j, 16)] = ws[j]
                wnv_v[pl.ds(16 * j, 16)] = ws[j] * rsv

            hoffs = [ehoff[p][pl.ds(16 * g, 16)] for g in range(4)]

            def dbody(d, carry):
                cos, nn = carry
                wv = jnp.full((16,), wnv_v[pl.ds(d, 16)][0])
                new_cos, new_nn = [], []
                for g in range(4):
                    v = plsc.load_gather(erows[p], [iota16 + 16 * g,
                                                    hoffs[g] + d])
                    new_cos.append(cos[g] + v * wv)
                    new_nn.append(nn[g] + v * v)
                return tuple(new_cos), tuple(new_nn)
            cos, nn = lax.fori_loop(0, DIM, dbody,
                                    ((zero16,) * 4, (zero16,) * 4), unroll=2)

            lgs = []
            for g in range(4):
                cg = cos[g] * _rsqrt16(jnp.maximum(nn[g], 1e-24))
                ppg = pp_v[pl.ds(c * ESTRIDE + 16 * g, 16)]
                eg = eid[p][pl.ds(16 * g, 16)]
                lg = ppg * s_w0 + cg * s_w1 + s_b
                lg = jnp.where(eg == 0, jnp.float32(-1e32), lg)
                if g == 3:
                    lg = jnp.where(iota16 >= ELEN - 48, jnp.float32(-jnp.inf), lg)
                lgs.append(lg)
            m = jnp.full((16,), jnp.max(jnp.maximum(jnp.maximum(lgs[0], lgs[1]),
                                                    jnp.maximum(lgs[2], lgs[3]))))
            es = [jnp.exp(lg - m) for lg in lgs]
            sv = jnp.full((16,), jnp.sum(es[0] + es[1] + es[2] + es[3]))
            invv = jnp.ones((16,), jnp.float32) / sv
            for g in range(4):
                att_v[pl.ds(16 * g, 16)] = es[g] * invv

            def pbody(e, faccs):
                av = jnp.full((16,), att_v[pl.ds(e, 16)][0])
                h = pl.multiple_of(ehoff[p][pl.ds(e, 16)][0], 64)
                return tuple(faccs[j] + erows[p][e, pl.ds(h + 16 * j, 16)] * av
                             for j in range(4))
            faccs = lax.fori_loop(0, ELEN, pbody, (zero16,) * 4, unroll=2)
            for j in range(4):
                feat_acc[c, pl.ds(16 * j, 16)] = faccs[j]

        fetch_idx(0, 0)
        fetch_idx(1, 1)
        wait_idx(0)
        prep(0)
        issue_gathers(0)

        def body(i, carry):
            for j in range(2):
                c = 2 * i + j
                p = j
                wait_gathers(p)

                @pl.when(c + 1 < NCH)
                def _():
                    wait_idx(1 - p)
                    prep(1 - p)
                    issue_gathers(1 - p)

                @pl.when(c + 2 < NCH)
                def _():
                    fetch_idx(c + 2, p)

                compute(c, p)
            return carry
        lax.fori_loop(0, NCH // 2, body, 0)

        pltpu.sync_copy(ws_acc, ws_out.at[pl.ds(woff, BPW)])
        pltpu.sync_copy(feat_acc, feat_out.at[pl.ds(woff, BPW)])

    return k(wt2, et2, widx_flat, eidx_flat, pp_flat, asc)


def _tc_body(ws_ref, fe_ref, wid_ref, owt_ref, ob_ref, o_ref):
    ws = ws_ref[:, :DIM]
    feat_e = fe_ref[:, :DIM]
    nz = jnp.sum((wid_ref[...] != 0).astype(jnp.float32), axis=1, keepdims=True)
    feat = feat_e + ws / nz
    o_ref[...] = (
        jnp.dot(feat, owt_ref[...], preferred_element_type=jnp.float32,
                precision=lax.Precision.HIGHEST)
        + ob_ref[...])


def _tc_out(ws128, feat128, word_ids, out_wt, out_b2):
    BB = 1024
    return pl.pallas_call(
        _tc_body,
        grid=(B // BB,),
        in_specs=[
            pl.BlockSpec((BB, 128), lambda i: (i, 0)),
            pl.BlockSpec((BB, 128), lambda i: (i, 0)),
            pl.BlockSpec((BB, WLEN), lambda i: (i, 0)),
            pl.BlockSpec((DIM, NUM_CLASSES), lambda i: (0, 0)),
            pl.BlockSpec((1, NUM_CLASSES), lambda i: (0, 0)),
        ],
        out_specs=pl.BlockSpec((BB, NUM_CLASSES), lambda i: (i, 0)),
        out_shape=jax.ShapeDtypeStruct((B, NUM_CLASSES), jnp.float32),
    )(ws128, feat128, word_ids, out_wt, out_b2)


def kernel(word_ids, entity_ids, prior_probs, word_table, entity_table,
           att_w, att_b, out_w, out_b):
    wt2 = word_table.reshape(500000, 128)
    et2 = entity_table.reshape(50000, 128)
    widx_flat = jnp.pad(word_ids, ((0, 0), (0, WSTRIDE - WLEN))).reshape(-1)
    eidx_flat = jnp.pad(entity_ids, ((0, 0), (0, ESTRIDE - ELEN))).reshape(-1)
    pp_flat = jnp.pad(prior_probs, ((0, 0), (0, ESTRIDE - ELEN))).reshape(-1)
    asc = jnp.concatenate([att_w.reshape(-1), att_b,
                           jnp.zeros(13, jnp.float32)])
    ws128, feat128 = _sc_forward(wt2, et2, widx_flat, eidx_flat, pp_flat, asc)
    return _tc_out(ws128, feat128, word_ids, out_w.T,
                   out_b.reshape(1, NUM_CLASSES))


# speedup vs baseline: 1.4272x; 1.4272x over previous
import functools

import jax
import jax.numpy as jnp
from jax import lax
from jax.experimental import pallas as pl
from jax.experimental.pallas import tpu as pltpu
from jax.experimental.pallas import tpu_sc as plsc

B = 4096
WLEN = 200
ELEN = 50
EPAD = 64
WSTRIDE = 256
ESTRIDE = 128
DIM = 64
NUM_CLASSES = 16

NC = 2
NS = 16
NW = NC * NS
BPW = B // NW
NPAIR = BPW // 2


def _sc_forward(wtab, etab, widx_flat, eidx_flat):
    mesh = plsc.VectorSubcoreMesh(core_axis_name="c", subcore_axis_name="s",
                                  num_cores=NC, num_subcores=NS)

    @functools.partial(
        pl.kernel,
        out_type=(
            jax.ShapeDtypeStruct((B // 2, 128), jnp.float32),
            jax.ShapeDtypeStruct((B // 2, EPAD, 128), jnp.float32),
        ),
        mesh=mesh,
        scratch_types=[
            pltpu.VMEM((2 * WSTRIDE,), jnp.int32),
            pltpu.VMEM((2 * WSTRIDE,), jnp.int32),
            pltpu.VMEM((2 * ESTRIDE,), jnp.int32),
            pltpu.VMEM((2 * ESTRIDE,), jnp.int32),
            pltpu.VMEM((2 * WLEN, DIM), jnp.float32),
            pltpu.VMEM((2 * WLEN, DIM), jnp.float32),
            pltpu.VMEM((2 * EPAD, DIM), jnp.float32),
            pltpu.VMEM((2 * EPAD, DIM), jnp.float32),
            pltpu.VMEM((EPAD, 128), jnp.float32),
            pltpu.VMEM((EPAD, 128), jnp.float32),
            pltpu.VMEM((NPAIR, 128), jnp.float32),
            pltpu.SemaphoreType.DMA,
            pltpu.SemaphoreType.DMA,
            pltpu.SemaphoreType.DMA,
            pltpu.SemaphoreType.DMA,
            pltpu.SemaphoreType.DMA,
            pltpu.SemaphoreType.DMA,
        ],
        compiler_params=pltpu.CompilerParams(use_tc_tiling_on_sc=False),
    )
    def k(wtab_h, etab_h, widx_h, eidx_h, ws_out, ev_out,
          widx0, widx1, eidx0, eidx1, wrows0, wrows1, erows0, erows1,
          pk0, pk1, ws_acc,
          semi0, semi1, semg0, semg1, semo0, semo1):
        w = lax.axis_index("s") * NC + lax.axis_index("c")
        poff = w * NPAIR
        widx = [widx0, widx1]
        eidx = [eidx0, eidx1]
        wrows = [wrows0, wrows1]
        erows = [erows0, erows1]
        pk = [pk0, pk1]
        semi = [semi0, semi1]
        semg = [semg0, semg1]
        semo = [semo0, semo1]
        zero16 = jnp.zeros((16,), jnp.float32)

        def fetch_idx(c, p):
            pltpu.async_copy(
                widx_h.at[pl.ds((poff + c) * 2 * WSTRIDE, 2 * WSTRIDE)],
                widx[p], semi[p])
            pltpu.async_copy(
                eidx_h.at[pl.ds((poff + c) * 2 * ESTRIDE, 2 * ESTRIDE)],
                eidx[p], semi[p])

        def wait_idx(p):
            pltpu.make_async_copy(widx_h.at[pl.ds(0, 2 * WSTRIDE)],
                                  widx[p], semi[p]).wait()
            pltpu.make_async_copy(eidx_h.at[pl.ds(0, 2 * ESTRIDE)],
                                  eidx[p], semi[p]).wait()

        def issue_gathers(p):
            for half, base in ((0, 0), (1, WSTRIDE)):
                pltpu.async_copy(
                    wtab_h.at[widx[p].at[pl.ds(base, 128)]],
                    wrows[p].at[pl.ds(half * WLEN, 128)], semg[p])
                pltpu.async_copy(
                    wtab_h.at[widx[p].at[pl.ds(base + 128, WLEN - 128)]],
                    wrows[p].at[pl.ds(half * WLEN + 128, WLEN - 128)], semg[p])
                pltpu.async_copy(
                    etab_h.at[eidx[p].at[pl.ds(half * ESTRIDE, EPAD)]],
                    erows[p].at[pl.ds(half * EPAD, EPAD)], semg[p])

        def wait_gathers(p):
            pltpu.make_async_copy(wtab_h.at[pl.ds(0, 2 * WLEN)],
                                  wrows[p], semg[p]).wait()
            pltpu.make_async_copy(etab_h.at[pl.ds(0, 2 * EPAD)],
                                  erows[p], semg[p]).wait()

        def compute(c, p):
            def rbody(r, accs):
                a, b2 = accs
                a = tuple(a[j] + wrows[p][r, pl.ds(16 * j, 16)]
                          for j in range(4))
                b2 = tuple(b2[j] + wrows[p][WLEN + r, pl.ds(16 * j, 16)]
                           for j in range(4))
                return (a, b2)
            acca, accb = lax.fori_loop(0, WLEN, rbody,
                                       ((zero16,) * 4, (zero16,) * 4),
                                       unroll=2)
            for j in range(4):
                ws_acc[c, pl.ds(16 * j, 16)] = acca[j]
                ws_acc[c, pl.ds(DIM + 16 * j, 16)] = accb[j]

            def ebody(e, carry):
                for j in range(4):
                    pk[p][e, pl.ds(16 * j, 16)] = \
                        erows[p][e, pl.ds(16 * j, 16)]
                    pk[p][e, pl.ds(DIM + 16 * j, 16)] = \
                        erows[p][EPAD + e, pl.ds(16 * j, 16)]
                return carry
            lax.fori_loop(0, EPAD, ebody, 0, unroll=2)
            pltpu.async_copy(pk[p], ev_out.at[poff + c], semo[p])

        def drain_out(p):
            pltpu.make_async_copy(pk[p], ev_out.at[0], semo[p]).wait()

        fetch_idx(0, 0)
        fetch_idx(1, 1)
        wait_idx(0)
        issue_gathers(0)

        def body(i, carry):
            for j in range(2):
                c = 2 * i + j
                p = j
                wait_gathers(p)

                @pl.when(c + 1 < NPAIR)
                def _():
                    wait_idx(1 - p)
                    issue_gathers(1 - p)

                @pl.when(c + 2 < NPAIR)
                def _():
                    fetch_idx(c + 2, p)

                @pl.when(c >= 2)
                def _():
                    drain_out(p)

                compute(c, p)
            return carry
        lax.fori_loop(0, NPAIR // 2, body, 0)
        drain_out(0)
        drain_out(1)
        pltpu.sync_copy(ws_acc, ws_out.at[pl.ds(poff, NPAIR)])

    return k(wtab, etab, widx_flat, eidx_flat)


def _tc_body(ws_ref, ev_ref, wid_ref, pp_ref, eid_ref, asc_ref, owt_ref,
             ob_ref, oa_ref, ob2_ref):
    lane = lax.broadcasted_iota(jnp.int32, (ws_ref.shape[0], EPAD), 1)
    for half, o_ref in ((0, oa_ref), (1, ob2_ref)):
        sl = slice(half * DIM, (half + 1) * DIM)
        ws = ws_ref[:, sl]
        ev = ev_ref[:, :, sl]
        pp = pp_ref[:, sl]
        eid = eid_ref[:, sl]
        wid = wid_ref[:, half * WLEN:(half + 1) * WLEN]
        wn = jnp.maximum(jnp.sqrt(jnp.sum(ws * ws, axis=1, keepdims=True)),
                         1e-12)
        wnv = ws / wn
        en = jnp.maximum(jnp.sqrt(jnp.sum(ev * ev, axis=2)), 1e-12)
        cos = jnp.sum(wnv[:, None, :] * ev, axis=2) / en
        lg = pp * asc_ref[0] + cos * asc_ref[1] + asc_ref[2]
        lg = jnp.where(eid == 0, jnp.float32(-1e32), lg)
        lg = jnp.where(lane >= ELEN, jnp.float32(-jnp.inf), lg)
        m = jnp.max(lg, axis=1, keepdims=True)
        e = jnp.exp(lg - m)
        att = e / jnp.sum(e, axis=1, keepdims=True)
        feat = jnp.sum(ev * att[:, :, None], axis=1)
        nz = jnp.sum((wid != 0).astype(jnp.float32), axis=1, keepdims=True)
        feat = feat + ws / nz
        o_ref[...] = (
            jnp.dot(feat, owt_ref[...], preferred_element_type=jnp.float32,
                    precision=lax.Precision.HIGHEST)
            + ob_ref[...])


def _tc_attn(ws_pair, ev_pair, wid_pair, pp_pair, eid_pair, att_scalars,
             out_wt, out_b2):
    BBH = 256
    grid = (B // 2 // BBH,)
    return pl.pallas_call(
        _tc_body,
        grid=grid,
        in_specs=[
            pl.BlockSpec((BBH, 128), lambda i: (i, 0)),
            pl.BlockSpec((BBH, EPAD, 128), lambda i: (i, 0, 0)),
            pl.BlockSpec((BBH, 2 * WLEN), lambda i: (i, 0)),
            pl.BlockSpec((BBH, 128), lambda i: (i, 0)),
            pl.BlockSpec((BBH, 128), lambda i: (i, 0)),
            pl.BlockSpec(memory_space=pltpu.SMEM),
            pl.BlockSpec((DIM, NUM_CLASSES), lambda i: (0, 0)),
            pl.BlockSpec((1, NUM_CLASSES), lambda i: (0, 0)),
        ],
        out_specs=[
            pl.BlockSpec((BBH, NUM_CLASSES), lambda i: (i, 0)),
            pl.BlockSpec((BBH, NUM_CLASSES), lambda i: (i, 0)),
        ],
        out_shape=(
            jax.ShapeDtypeStruct((B // 2, NUM_CLASSES), jnp.float32),
            jax.ShapeDtypeStruct((B // 2, NUM_CLASSES), jnp.float32),
        ),
    )(ws_pair, ev_pair, wid_pair, pp_pair, eid_pair, att_scalars,
      out_wt, out_b2)


def kernel(word_ids, entity_ids, prior_probs, word_table, entity_table,
           att_w, att_b, out_w, out_b):
    widx_flat = jnp.pad(word_ids, ((0, 0), (0, WSTRIDE - WLEN))).reshape(-1)
    eidx_flat = jnp.pad(entity_ids, ((0, 0), (0, ESTRIDE - ELEN))).reshape(-1)
    ws_pair, ev_pair = _sc_forward(word_table, entity_table, widx_flat,
                                   eidx_flat)
    wid_pair = word_ids.reshape(B // 2, 2 * WLEN)
    pp_pair = jnp.pad(prior_probs,
                      ((0, 0), (0, EPAD - ELEN))).reshape(B // 2, 128)
    eid_pair = jnp.pad(entity_ids,
                       ((0, 0), (0, EPAD - ELEN))).reshape(B // 2, 128)
    asc = jnp.stack([att_w[0, 0], att_w[0, 1], att_b[0]])
    oa, ob2 = _tc_attn(ws_pair, ev_pair, wid_pair, pp_pair, eid_pair, asc,
                       out_w.T, out_b.reshape(1, NUM_CLASSES))
    return jnp.stack([oa, ob2], axis=1).reshape(B, NUM_CLASSES)


# speedup vs baseline: 1.4287x; 1.0011x over previous
import functools

import jax
import jax.numpy as jnp
from jax import lax
from jax.experimental import pallas as pl
from jax.experimental.pallas import tpu as pltpu
from jax.experimental.pallas import tpu_sc as plsc

B = 4096
WLEN = 200
ELEN = 50
EPAD = 64
WSTRIDE = 256
ESTRIDE = 128
DIM = 64
NUM_CLASSES = 16

NC = 2
NS = 16
NW = NC * NS
BPW = B // NW
CB = 4
NCHUNK = BPW // CB


def _sc_forward(wtab, etab, widx_flat, eidx_flat):
    mesh = plsc.VectorSubcoreMesh(core_axis_name="c", subcore_axis_name="s",
                                  num_cores=NC, num_subcores=NS)

    @functools.partial(
        pl.kernel,
        out_type=(
            jax.ShapeDtypeStruct((B // 2, 128), jnp.float32),
            jax.ShapeDtypeStruct((B // 2, EPAD, 128), jnp.float32),
        ),
        mesh=mesh,
        scratch_types=[
            pltpu.VMEM((CB * WSTRIDE,), jnp.int32),
            pltpu.VMEM((CB * ESTRIDE,), jnp.int32),
            pltpu.VMEM((CB * WLEN, DIM), jnp.float32),
            pltpu.VMEM((CB * EPAD, DIM), jnp.float32),
            pltpu.VMEM((CB // 2, EPAD, 128), jnp.float32),
            pltpu.VMEM((BPW // 2, 128), jnp.float32),
            pltpu.SemaphoreType.DMA,
        ],
        compiler_params=pltpu.CompilerParams(use_tc_tiling_on_sc=False),
    )
    def k(wtab_h, etab_h, widx_h, eidx_h, ws_out, ev_out,
          widx_v, eidx_v, wrows_v, erows_v, pk_v, ws_acc, sem):
        w = lax.axis_index("s") * NC + lax.axis_index("c")
        zero16 = jnp.zeros((16,), jnp.float32)

        def chunk(c, carry):
            base = w * BPW + c * CB
            gp = (w * BPW) // 2 + c * (CB // 2)
            pltpu.sync_copy(widx_h.at[pl.ds(base * WSTRIDE, CB * WSTRIDE)],
                            widx_v)
            pltpu.sync_copy(eidx_h.at[pl.ds(base * ESTRIDE, CB * ESTRIDE)],
                            eidx_v)
            cps = []
            for cc in range(CB):
                cps.append(pltpu.async_copy(
                    wtab_h.at[widx_v.at[pl.ds(cc * WSTRIDE, 128)]],
                    wrows_v.at[pl.ds(cc * WLEN, 128)], sem))
                cps.append(pltpu.async_copy(
                    wtab_h.at[widx_v.at[pl.ds(cc * WSTRIDE + 128, WLEN - 128)]],
                    wrows_v.at[pl.ds(cc * WLEN + 128, WLEN - 128)], sem))
                cps.append(pltpu.async_copy(
                    etab_h.at[eidx_v.at[pl.ds(cc * ESTRIDE, EPAD)]],
                    erows_v.at[pl.ds(cc * EPAD, EPAD)], sem))
            for cp in cps:
                cp.wait()

            def rbody(r, accs):
                return tuple(accs[cc * 4 + j]
                             + wrows_v[cc * WLEN + r, pl.ds(16 * j, 16)]
                             for cc in range(CB) for j in range(4))
            accs = lax.fori_loop(0, WLEN, rbody, (zero16,) * (CB * 4))
            for cc in range(CB):
                for j in range(4):
                    ws_acc[c * (CB // 2) + cc // 2,
                           pl.ds((cc % 2) * DIM + 16 * j, 16)] = accs[cc * 4 + j]

            def ebody(e, carry2):
                for q in range(CB // 2):
                    for j in range(4):
                        pk_v[q, e, pl.ds(16 * j, 16)] = \
                            erows_v[2 * q * EPAD + e, pl.ds(16 * j, 16)]
                        pk_v[q, e, pl.ds(DIM + 16 * j, 16)] = \
                            erows_v[(2 * q + 1) * EPAD + e, pl.ds(16 * j, 16)]
                return carry2
            lax.fori_loop(0, EPAD, ebody, 0)
            pltpu.sync_copy(pk_v, ev_out.at[pl.ds(gp, CB // 2)])
            return carry

        lax.fori_loop(0, NCHUNK, chunk, 0)
        pltpu.sync_copy(ws_acc, ws_out.at[pl.ds((w * BPW) // 2, BPW // 2)])

    return k(wtab, etab, widx_flat, eidx_flat)


def _tc_body(ws_ref, ev_ref, wid_ref, pp_ref, eid_ref, asc_ref, owt_ref,
             ob_ref, oa_ref, ob2_ref):
    lane = lax.broadcasted_iota(jnp.int32, (ws_ref.shape[0], EPAD), 1)
    for half, o_ref in ((0, oa_ref), (1, ob2_ref)):
        sl = slice(half * DIM, (half + 1) * DIM)
        ws = ws_ref[:, sl]
        ev = ev_ref[:, :, sl]
        pp = pp_ref[:, sl]
        eid = eid_ref[:, sl]
        wid = wid_ref[:, half * WLEN:(half + 1) * WLEN]
        wn = jnp.maximum(jnp.sqrt(jnp.sum(ws * ws, axis=1, keepdims=True)),
                         1e-12)
        wnv = ws / wn
        en = jnp.maximum(jnp.sqrt(jnp.sum(ev * ev, axis=2)), 1e-12)
        cos = jnp.sum(wnv[:, None, :] * ev, axis=2) / en
        lg = pp * asc_ref[0] + cos * asc_ref[1] + asc_ref[2]
        lg = jnp.where(eid == 0, jnp.float32(-1e32), lg)
        lg = jnp.where(lane >= ELEN, jnp.float32(-jnp.inf), lg)
        m = jnp.max(lg, axis=1, keepdims=True)
        e = jnp.exp(lg - m)
        att = e / jnp.sum(e, axis=1, keepdims=True)
        feat = jnp.sum(ev * att[:, :, None], axis=1)
        nz = jnp.sum((wid != 0).astype(jnp.float32), axis=1, keepdims=True)
        feat = feat + ws / nz
        o_ref[...] = (
            jnp.dot(feat, owt_ref[...], preferred_element_type=jnp.float32,
                    precision=lax.Precision.HIGHEST)
            + ob_ref[...])


def _tc_attn(ws_pair, ev_pair, wid_pair, pp_pair, eid_pair, att_scalars,
             out_wt, out_b2):
    BBH = 256
    grid = (B // 2 // BBH,)
    return pl.pallas_call(
        _tc_body,
        grid=grid,
        in_specs=[
            pl.BlockSpec((BBH, 128), lambda i: (i, 0)),
            pl.BlockSpec((BBH, EPAD, 128), lambda i: (i, 0, 0)),
            pl.BlockSpec((BBH, 2 * WLEN), lambda i: (i, 0)),
            pl.BlockSpec((BBH, 128), lambda i: (i, 0)),
            pl.BlockSpec((BBH, 128), lambda i: (i, 0)),
            pl.BlockSpec(memory_space=pltpu.SMEM),
            pl.BlockSpec((DIM, NUM_CLASSES), lambda i: (0, 0)),
            pl.BlockSpec((1, NUM_CLASSES), lambda i: (0, 0)),
        ],
        out_specs=[
            pl.BlockSpec((BBH, NUM_CLASSES), lambda i: (i, 0)),
            pl.BlockSpec((BBH, NUM_CLASSES), lambda i: (i, 0)),
        ],
        out_shape=(
            jax.ShapeDtypeStruct((B // 2, NUM_CLASSES), jnp.float32),
            jax.ShapeDtypeStruct((B // 2, NUM_CLASSES), jnp.float32),
        ),
    )(ws_pair, ev_pair, wid_pair, pp_pair, eid_pair, att_scalars,
      out_wt, out_b2)


def kernel(word_ids, entity_ids, prior_probs, word_table, entity_table,
           att_w, att_b, out_w, out_b):
    widx_flat = jnp.pad(word_ids, ((0, 0), (0, WSTRIDE - WLEN))).reshape(-1)
    eidx_flat = jnp.pad(entity_ids, ((0, 0), (0, ESTRIDE - ELEN))).reshape(-1)
    ws_pair, ev_pair = _sc_forward(word_table, entity_table, widx_flat,
                                   eidx_flat)
    wid_pair = word_ids.reshape(B // 2, 2 * WLEN)
    pp_pair = jnp.pad(prior_probs,
                      ((0, 0), (0, EPAD - ELEN))).reshape(B // 2, 128)
    eid_pair = jnp.pad(entity_ids,
                       ((0, 0), (0, EPAD - ELEN))).reshape(B // 2, 128)
    asc = jnp.stack([att_w[0, 0], att_w[0, 1], att_b[0]])
    oa, ob2 = _tc_attn(ws_pair, ev_pair, wid_pair, pp_pair, eid_pair, asc,
                       out_w.T, out_b.reshape(1, NUM_CLASSES))
    return jnp.stack([oa, ob2], axis=1).reshape(B, NUM_CLASSES)
